# baseline (speedup 1.0000x reference)
import functools

import jax
import jax.numpy as jnp
from jax import lax
from jax.experimental import pallas as pl
from jax.experimental.pallas import tpu as pltpu
from jax.experimental.pallas import tpu_sc as plsc

_B, _D, _P, _C, _T = 1024, 1024, 500, 1000, 20000
_NC, _NS, _L = 2, 16, 16
_NW = _NC * _NS
_ROWS = _B // _NW
_TB = 800
_NCHUNK = _T // _TB
_NV = _TB // _L


def _mm_body(feat_ref, wt_ref, b_ref, out_ref):
    out_ref[...] = (
        jnp.dot(feat_ref[...], wt_ref[...], preferred_element_type=jnp.float32)
        + b_ref[...]
    )


def _linear(feat, W, b):
    return pl.pallas_call(
        _mm_body,
        out_shape=jax.ShapeDtypeStruct((_B, _P), jnp.float32),
    )(feat, W.T, b.reshape(1, _P))


def _gather_body(ps_hbm, p_hbm, po_hbm, ss_hbm, sp_hbm, so_hbm, out_hbm,
                 s_tile, p_tile, o_tile, idx_s, idx_p, idx_o, out_buf):
    wid = lax.axis_index("s") * _NC + lax.axis_index("c")
    rbase = wid * _ROWS
    pltpu.sync_copy(ps_hbm.at[pl.ds(rbase, _ROWS), :], s_tile)
    pltpu.sync_copy(p_hbm.at[pl.ds(rbase, _ROWS), :], p_tile)
    pltpu.sync_copy(po_hbm.at[pl.ds(rbase, _ROWS), :], o_tile)

    @pl.loop(0, _NCHUNK)
    def _chunk(ci):
        tbase = ci * _TB
        pltpu.sync_copy(ss_hbm.at[pl.ds(tbase, _TB)], idx_s)
        pltpu.sync_copy(sp_hbm.at[pl.ds(tbase, _TB)], idx_p)
        pltpu.sync_copy(so_hbm.at[pl.ds(tbase, _TB)], idx_o)

        @pl.loop(0, _NV)
        def _vec(v):
            off = v * _L
            cs = idx_s[pl.ds(off, _L)]
            cp = idx_p[pl.ds(off, _L)]
            co = idx_o[pl.ds(off, _L)]
            for r in range(_ROWS):
                rv = jnp.full((_L,), r, jnp.int32)
                sv = plsc.load_gather(s_tile, [rv, cs])
                pv = plsc.load_gather(p_tile, [rv, cp])
                ov = plsc.load_gather(o_tile, [rv, co])
                out_buf[r, pl.ds(off, _L)] = sv * pv * ov

        pltpu.sync_copy(out_buf,
                        out_hbm.at[pl.ds(rbase, _ROWS), pl.ds(tbase, _TB)])


_gather = functools.partial(
    pl.kernel,
    out_type=jax.ShapeDtypeStruct((_B, _T), jnp.float32),
    mesh=plsc.VectorSubcoreMesh(
        core_axis_name="c", subcore_axis_name="s",
        num_cores=_NC, num_subcores=_NS),
    compiler_params=pltpu.CompilerParams(
        use_tc_tiling_on_sc=False, needs_layout_passes=False),
    scratch_types=[
        pltpu.VMEM((_ROWS, _C), jnp.float32),
        pltpu.VMEM((_ROWS, _P), jnp.float32),
        pltpu.VMEM((_ROWS, _C), jnp.float32),
        pltpu.VMEM((_TB,), jnp.int32),
        pltpu.VMEM((_TB,), jnp.int32),
        pltpu.VMEM((_TB,), jnp.int32),
        pltpu.VMEM((_ROWS, _TB), jnp.float32),
    ],
)(_gather_body)


def kernel(feat, prob_s, prob_o, W, b, sel_s, sel_p, sel_o):
    p = _linear(feat, W, b)
    return _gather(prob_s, p, prob_o, sel_s, sel_p, sel_o)


# speedup vs baseline: 1.1555x; 1.1555x over previous
import functools

import jax
import jax.numpy as jnp
from jax import lax
from jax.experimental import pallas as pl
from jax.experimental.pallas import tpu as pltpu
from jax.experimental.pallas import tpu_sc as plsc

_B, _D, _P, _C, _T = 1024, 1024, 500, 1000, 20000
_NC, _NS, _L = 2, 16, 16
_NW = _NC * _NS
_ROWS = _B // _NW
_TB = 400
_NCHUNK = _T // _TB
_NV = _TB // _L


def _mm_body(feat_ref, wt_ref, b_ref, out_ref):
    out_ref[...] = (
        jnp.dot(feat_ref[...], wt_ref[...], preferred_element_type=jnp.float32)
        + b_ref[...]
    )


def _linear(feat, W, b):
    return pl.pallas_call(
        _mm_body,
        out_shape=jax.ShapeDtypeStruct((_B, _P), jnp.float32),
    )(feat, W.T, b.reshape(1, _P))


def _gather_body(ps_hbm, p_hbm, po_hbm, ss_hbm, sp_hbm, so_hbm, out_hbm,
                 s_tile, p_tile, o_tile,
                 idx_s0, idx_p0, idx_o0, idx_s1, idx_p1, idx_o1,
                 out0, out1, sem_i0, sem_i1, sem_o0, sem_o1):
    wid = lax.axis_index("s") * _NC + lax.axis_index("c")
    rbase = wid * _ROWS
    slots = ((idx_s0, idx_p0, idx_o0, out0, sem_i0, sem_o0),
             (idx_s1, idx_p1, idx_o1, out1, sem_i1, sem_o1))

    def idx_start(c, slot):
        idx_s, idx_p, idx_o, _, sem_i, _ = slot
        tbase = c * _TB
        pltpu.async_copy(ss_hbm.at[pl.ds(tbase, _TB)], idx_s, sem_i)
        pltpu.async_copy(sp_hbm.at[pl.ds(tbase, _TB)], idx_p, sem_i)
        pltpu.async_copy(so_hbm.at[pl.ds(tbase, _TB)], idx_o, sem_i)

    def idx_wait(slot):
        idx_s, idx_p, idx_o, _, sem_i, _ = slot
        pltpu.make_async_copy(ss_hbm.at[pl.ds(0, _TB)], idx_s, sem_i).wait()
        pltpu.make_async_copy(sp_hbm.at[pl.ds(0, _TB)], idx_p, sem_i).wait()
        pltpu.make_async_copy(so_hbm.at[pl.ds(0, _TB)], idx_o, sem_i).wait()

    def compute(slot):
        idx_s, idx_p, idx_o, obuf, _, _ = slot

        @pl.loop(0, _NV)
        def _vec(v):
            off = v * _L
            cs = idx_s[pl.ds(off, _L)]
            cp = idx_p[pl.ds(off, _L)]
            co = idx_o[pl.ds(off, _L)]
            for r in range(_ROWS):
                rv = jnp.full((_L,), r, jnp.int32)
                sv = plsc.load_gather(s_tile, [rv, cs])
                pv = plsc.load_gather(p_tile, [rv, cp])
                ov = plsc.load_gather(o_tile, [rv, co])
                obuf[r, pl.ds(off, _L)] = sv * pv * ov

    def out_start(c, slot):
        _, _, _, obuf, _, sem_o = slot
        pltpu.async_copy(
            obuf, out_hbm.at[pl.ds(rbase, _ROWS), pl.ds(c * _TB, _TB)], sem_o)

    def out_wait(slot):
        _, _, _, obuf, _, sem_o = slot
        pltpu.make_async_copy(
            obuf, out_hbm.at[pl.ds(rbase, _ROWS), pl.ds(0, _TB)], sem_o).wait()

    pltpu.sync_copy(ps_hbm.at[pl.ds(rbase, _ROWS), :], s_tile)
    pltpu.sync_copy(p_hbm.at[pl.ds(rbase, _ROWS), :], p_tile)
    pltpu.sync_copy(po_hbm.at[pl.ds(rbase, _ROWS), :], o_tile)

    idx_start(0, slots[0])
    idx_start(1, slots[1])
    for k in range(2):
        idx_wait(slots[k])
        compute(slots[k])
        out_start(k, slots[k])
        idx_start(k + 2, slots[k])

    @pl.loop(2, _NCHUNK - 2, step=2)
    def _chunk(ci):
        for k in range(2):
            c = ci + k
            out_wait(slots[k])
            idx_wait(slots[k])
            compute(slots[k])
            out_start(c, slots[k])
            idx_start(c + 2, slots[k])

    for k in range(2):
        c = _NCHUNK - 2 + k
        out_wait(slots[k])
        idx_wait(slots[k])
        compute(slots[k])
        out_start(c, slots[k])
    for k in range(2):
        out_wait(slots[k])


_gather = functools.partial(
    pl.kernel,
    out_type=jax.ShapeDtypeStruct((_B, _T), jnp.float32),
    mesh=plsc.VectorSubcoreMesh(
        core_axis_name="c", subcore_axis_name="s",
        num_cores=_NC, num_subcores=_NS),
    compiler_params=pltpu.CompilerParams(
        use_tc_tiling_on_sc=False, needs_layout_passes=False),
    scratch_types=[
        pltpu.VMEM((_ROWS, _C), jnp.float32),
        pltpu.VMEM((_ROWS, _P), jnp.float32),
        pltpu.VMEM((_ROWS, _C), jnp.float32),
        pltpu.VMEM((_TB,), jnp.int32),
        pltpu.VMEM((_TB,), jnp.int32),
        pltpu.VMEM((_TB,), jnp.int32),
        pltpu.VMEM((_TB,), jnp.int32),
        pltpu.VMEM((_TB,), jnp.int32),
        pltpu.VMEM((_TB,), jnp.int32),
        pltpu.VMEM((_ROWS, _TB), jnp.float32),
        pltpu.VMEM((_ROWS, _TB), jnp.float32),
        pltpu.SemaphoreType.DMA,
        pltpu.SemaphoreType.DMA,
        pltpu.SemaphoreType.DMA,
        pltpu.SemaphoreType.DMA,
    ],
)(_gather_body)


def kernel(feat, prob_s, prob_o, W, b, sel_s, sel_p, sel_o):
    p = _linear(feat, W, b)
    return _gather(prob_s, p, prob_o, sel_s, sel_p, sel_o)


# speedup vs baseline: 1.9140x; 1.6564x over previous
import functools

import jax
import jax.numpy as jnp
from jax import lax
from jax.experimental import pallas as pl
from jax.experimental.pallas import tpu as pltpu
from jax.experimental.pallas import tpu_sc as plsc

_B, _D, _P, _C, _T = 1024, 1024, 500, 1000, 20000
_NC, _NS, _L = 2, 16, 16
_NW = _NC * _NS
_ROWS = _B // _NW
_TB = 640
_NF = _T // _TB
_RB = _T - _NF * _TB
_NV = _TB // _L
_RV = _RB // _L


def _mm_body(feat_ref, wt_ref, b_ref, out_ref):
    out_ref[...] = (
        jnp.dot(feat_ref[...], wt_ref[...], preferred_element_type=jnp.float32)
        + b_ref[...]
    )


def _linear(feat, W, b):
    return pl.pallas_call(
        _mm_body,
        out_shape=jax.ShapeDtypeStruct((_B, _P), jnp.float32),
    )(feat, W.T, b.reshape(1, _P))


def _gather_body(ps_hbm, p_hbm, po_hbm, ss_hbm, sp_hbm, so_hbm, out_hbm,
                 s_tile, p_tile, o_tile,
                 idx_s0, idx_p0, idx_o0, idx_s1, idx_p1, idx_o1,
                 out0, out1, sem_i0, sem_i1, sem_o0, sem_o1):
    wid = lax.axis_index("s") * _NC + lax.axis_index("c")
    rbase = wid * _ROWS
    slots = ((idx_s0, idx_p0, idx_o0, out0, sem_i0, sem_o0),
             (idx_s1, idx_p1, idx_o1, out1, sem_i1, sem_o1))

    def idx_start(c, slot):
        idx_s, idx_p, idx_o, _, sem_i, _ = slot
        tbase = c * _TB
        pltpu.async_copy(ss_hbm.at[pl.ds(tbase, _TB)], idx_s, sem_i)
        pltpu.async_copy(sp_hbm.at[pl.ds(tbase, _TB)], idx_p, sem_i)
        pltpu.async_copy(so_hbm.at[pl.ds(tbase, _TB)], idx_o, sem_i)

    def idx_wait(slot):
        idx_s, idx_p, idx_o, _, sem_i, _ = slot
        pltpu.make_async_copy(ss_hbm.at[pl.ds(0, _TB)], idx_s, sem_i).wait()
        pltpu.make_async_copy(sp_hbm.at[pl.ds(0, _TB)], idx_p, sem_i).wait()
        pltpu.make_async_copy(so_hbm.at[pl.ds(0, _TB)], idx_o, sem_i).wait()

    def compute(slot, nv):
        idx_s, idx_p, idx_o, obuf, _, _ = slot

        @plsc.parallel_loop(0, nv)
        def _vec(v):
            off = v * _L
            cs = idx_s[pl.ds(off, _L)]
            cp = idx_p[pl.ds(off, _L)]
            co = idx_o[pl.ds(off, _L)]
            for r in range(_ROWS):
                rv = jnp.full((_L,), r, jnp.int32)
                sv = plsc.load_gather(s_tile, [rv, cs])
                pv = plsc.load_gather(p_tile, [rv, cp])
                ov = plsc.load_gather(o_tile, [rv, co])
                obuf[r, pl.ds(off, _L)] = sv * pv * ov

    def out_start(c, slot):
        _, _, _, obuf, _, sem_o = slot
        pltpu.async_copy(
            obuf, out_hbm.at[pl.ds(rbase, _ROWS), pl.ds(c * _TB, _TB)], sem_o)

    def out_wait(slot):
        _, _, _, obuf, _, sem_o = slot
        pltpu.make_async_copy(
            obuf, out_hbm.at[pl.ds(rbase, _ROWS), pl.ds(0, _TB)], sem_o).wait()

    pltpu.sync_copy(ps_hbm.at[pl.ds(rbase, _ROWS), :], s_tile)
    pltpu.sync_copy(p_hbm.at[pl.ds(rbase, _ROWS), :], p_tile)
    pltpu.sync_copy(po_hbm.at[pl.ds(rbase, _ROWS), :], o_tile)

    idx_start(0, slots[0])
    idx_start(1, slots[1])
    for k in range(2):
        idx_wait(slots[k])
        compute(slots[k], _NV)
        out_start(k, slots[k])
        idx_start(k + 2, slots[k])

    @pl.loop(2, _NF - 3, step=2)
    def _chunk(ci):
        for k in range(2):
            c = ci + k
            out_wait(slots[k])
            idx_wait(slots[k])
            compute(slots[k], _NV)
            out_start(c, slots[k])
            idx_start(c + 2, slots[k])

    for k in range(2):
        c = _NF - 3 + k
        out_wait(slots[k])
        idx_wait(slots[k])
        compute(slots[k], _NV)
        out_start(c, slots[k])
        if c + 2 < _NF:
            idx_start(c + 2, slots[k])

    out_wait(slots[0])
    idx_wait(slots[0])
    compute(slots[0], _NV)
    out_start(_NF - 1, slots[0])

    out_wait(slots[1])
    rem = _NF * _TB
    pltpu.sync_copy(ss_hbm.at[pl.ds(rem, _RB)], idx_s1.at[pl.ds(0, _RB)])
    pltpu.sync_copy(sp_hbm.at[pl.ds(rem, _RB)], idx_p1.at[pl.ds(0, _RB)])
    pltpu.sync_copy(so_hbm.at[pl.ds(rem, _RB)], idx_o1.at[pl.ds(0, _RB)])
    compute(slots[1], _RV)
    pltpu.sync_copy(out1.at[:, pl.ds(0, _RB)],
                    out_hbm.at[pl.ds(rbase, _ROWS), pl.ds(rem, _RB)])
    out_wait(slots[0])


_gather = functools.partial(
    pl.kernel,
    out_type=jax.ShapeDtypeStruct((_B, _T), jnp.float32),
    mesh=plsc.VectorSubcoreMesh(
        core_axis_name="c", subcore_axis_name="s",
        num_cores=_NC, num_subcores=_NS),
    compiler_params=pltpu.CompilerParams(needs_layout_passes=False),
    scratch_types=[
        pltpu.VMEM((_ROWS, _C), jnp.float32),
        pltpu.VMEM((_ROWS, _P), jnp.float32),
        pltpu.VMEM((_ROWS, _C), jnp.float32),
        pltpu.VMEM((_TB,), jnp.int32),
        pltpu.VMEM((_TB,), jnp.int32),
        pltpu.VMEM((_TB,), jnp.int32),
        pltpu.VMEM((_TB,), jnp.int32),
        pltpu.VMEM((_TB,), jnp.int32),
        pltpu.VMEM((_TB,), jnp.int32),
        pltpu.VMEM((_ROWS, _TB), jnp.float32),
        pltpu.VMEM((_ROWS, _TB), jnp.float32),
        pltpu.SemaphoreType.DMA,
        pltpu.SemaphoreType.DMA,
        pltpu.SemaphoreType.DMA,
        pltpu.SemaphoreType.DMA,
    ],
)(_gather_body)


def kernel(feat, prob_s, prob_o, W, b, sel_s, sel_p, sel_o):
    p = _linear(feat, W, b)
    return _gather(prob_s, p, prob_o, sel_s, sel_p, sel_o)
